# Initial kernel scaffold; baseline (speedup 1.0000x reference)
import jax
import jax.numpy as jnp
from jax.experimental import pallas as pl


def kernel(x, bn1_g, bn1_b, bn1_m, bn1_v, conv1_w, conv1_b, bn2_g, bn2_b, bn2_m, bn2_v, conv2_w, conv2_b, bn3_g, bn3_b, bn3_m, bn3_v, conv3_w, conv3_b, se_fc1_w, se_fc1_b, se_fc2_w, se_fc2_b):
    raise NotImplementedError("write your pallas kernel here")


# speedup vs baseline: 1.2789x; 1.2789x over previous
import jax
import jax.numpy as jnp
from jax.experimental import pallas as pl
from jax.experimental.pallas import tpu as pltpu


def _swish(x):
    return x * (1.0 / (1.0 + jnp.exp(-x)))


def _bn_affine(gamma, beta, mean, var, eps):
    s = gamma / jnp.sqrt(var + eps)
    return s, beta - mean * s


def _col(v):
    return v.reshape(-1, 1).astype(jnp.float32)


def _make_fused_kernel(nb, mid, cout, cin, lq, l_out, left, L, K):
    inv_l = 1.0 / float(l_out)
    lc = (cout - cin) // 2
    rc = cout - cin - lc

    def body(x_ref, s1_ref, t1_ref, w1_ref, s2_ref, t2_ref, w2_ref,
             s3_ref, t3_ref, w3_ref, b3_ref, wf1_ref, bf1_ref,
             wf2_ref, bf2_ref, o_ref):
        s1 = s1_ref[...]
        t1 = t1_ref[...]
        w1 = w1_ref[...]
        s2 = s2_ref[...]
        t2 = t2_ref[...]
        w2 = w2_ref[...]
        s3 = s3_ref[...]
        t3 = t3_ref[...]
        w3 = w3_ref[...]
        b3 = b3_ref[...]
        wf1 = wf1_ref[...]
        bf1 = bf1_ref[...]
        wf2 = wf2_ref[...]
        bf2 = bf2_ref[...]
        col = jax.lax.broadcasted_iota(jnp.int32, (mid, lq), 1)
        masks = [((2 * col + s >= left) & (2 * col + s < left + L))
                 for s in range(2)]
        zl = jnp.zeros((lc, l_out), jnp.float32)
        zr = jnp.zeros((rc, l_out), jnp.float32)
        for i in range(nb):
            xp = [x_ref[i, s] for s in range(2)]
            h = []
            for s in range(2):
                a = _swish(s1 * xp[s] + t1)
                y = jnp.dot(w1, a, preferred_element_type=jnp.float32)
                h.append(jnp.where(masks[s], _swish(s2 * y + t2), 0.0))
            slab = jnp.concatenate(
                [h[k % 2][:, (k // 2):(k // 2) + l_out] for k in range(K)],
                axis=0)
            y2 = jnp.dot(w2, slab, preferred_element_type=jnp.float32)
            h3 = _swish(s3 * y2 + t3)
            y3 = jnp.dot(w3, h3, preferred_element_type=jnp.float32) + b3
            se = jnp.sum(y3, axis=-1, keepdims=True) * inv_l
            se_b = jnp.broadcast_to(se, (cout, 128))
            z1 = _swish(jnp.dot(wf1, se_b, preferred_element_type=jnp.float32) + bf1)
            z2 = jnp.dot(wf2, z1, preferred_element_type=jnp.float32) + bf2
            gate = (1.0 / (1.0 + jnp.exp(-z2)))[:, 0:1]
            ident = jnp.maximum(xp[1][:, 0:l_out], xp[0][:, 1:1 + l_out])
            idp = jnp.concatenate([zl, ident, zr], axis=0)
            o_ref[i] = (y3 * gate + idp).astype(o_ref.dtype)
    return body


def kernel(x, bn1_g, bn1_b, bn1_m, bn1_v, conv1_w, conv1_b,
           bn2_g, bn2_b, bn2_m, bn2_v, conv2_w, conv2_b,
           bn3_g, bn3_b, bn3_m, bn3_v, conv3_w, conv3_b,
           se_fc1_w, se_fc1_b, se_fc2_w, se_fc2_b):
    K, stride, groups = 5, 2, 2
    bn_eps = 1e-5
    N, Cin, L = x.shape
    mid = conv1_w.shape[0]
    Cout = conv3_w.shape[0]
    half = se_fc1_w.shape[0]
    cin_g = mid // groups

    s1, t1 = _bn_affine(bn1_g, bn1_b, bn1_m, bn1_v, bn_eps)
    s2, t2 = _bn_affine(bn2_g, bn2_b, bn2_m, bn2_v, bn_eps)
    s3, t3 = _bn_affine(bn3_g, bn3_b, bn3_m, bn3_v, bn_eps)
    t2 = t2 + s2 * conv1_b
    t3 = t3 + s3 * conv2_b

    L_out = -(-L // stride)
    p = max(0, (L_out - 1) * stride + K - L)
    left = p // 2
    right = p - left
    Lq = -(-(L + p) // stride)
    extra = Lq * stride - (L + p)
    assert left == 1 and L % 2 == 0 and stride == 2

    w2f = conv2_w.astype(jnp.float32)
    w2b = jnp.zeros((mid, K, mid), jnp.float32)
    for g in range(groups):
        c0 = g * cin_g
        w2b = w2b.at[c0:c0 + cin_g, :, c0:c0 + cin_g].set(
            w2f[c0:c0 + cin_g].transpose(0, 2, 1))
    w2b = w2b.reshape(mid, K * mid)

    xf = x.astype(jnp.float32)
    x_pad = jnp.pad(xf, ((0, 0), (0, 0), (left, right + extra)))
    x_ph = x_pad.reshape(N, Cin, Lq, stride).transpose(0, 3, 1, 2)

    nb = next(c for c in (8, 4, 2, 1) if N % c == 0)
    grid = (N // nb,)
    bs = pl.BlockSpec

    out = pl.pallas_call(
        _make_fused_kernel(nb, mid, Cout, Cin, Lq, L_out, left, L, K),
        out_shape=jax.ShapeDtypeStruct((N, Cout, L_out), jnp.float32),
        grid=grid,
        in_specs=[
            bs((nb, stride, Cin, Lq), lambda n: (n, 0, 0, 0)),
            bs((Cin, 1), lambda n: (0, 0)),
            bs((Cin, 1), lambda n: (0, 0)),
            bs((mid, Cin), lambda n: (0, 0)),
            bs((mid, 1), lambda n: (0, 0)),
            bs((mid, 1), lambda n: (0, 0)),
            bs((mid, K * mid), lambda n: (0, 0)),
            bs((mid, 1), lambda n: (0, 0)),
            bs((mid, 1), lambda n: (0, 0)),
            bs((Cout, mid), lambda n: (0, 0)),
            bs((Cout, 1), lambda n: (0, 0)),
            bs((half, Cout), lambda n: (0, 0)),
            bs((half, 1), lambda n: (0, 0)),
            bs((Cout, half), lambda n: (0, 0)),
            bs((Cout, 1), lambda n: (0, 0)),
        ],
        out_specs=bs((nb, Cout, L_out), lambda n: (n, 0, 0)),
        compiler_params=pltpu.CompilerParams(
            dimension_semantics=("parallel",)),
    )(x_ph, _col(s1), _col(t1), conv1_w[:, :, 0].astype(jnp.float32),
      _col(s2), _col(t2), w2b, _col(s3), _col(t3),
      conv3_w[:, :, 0].astype(jnp.float32), _col(conv3_b),
      se_fc1_w.astype(jnp.float32), _col(se_fc1_b),
      se_fc2_w.astype(jnp.float32), _col(se_fc2_b))
    return out
